# Initial kernel scaffold; baseline (speedup 1.0000x reference)
import jax
import jax.numpy as jnp
from jax.experimental import pallas as pl


def kernel(x, embed):
    raise NotImplementedError("write your pallas kernel here")


# speedup vs baseline: 5.1194x; 5.1194x over previous
import jax
import jax.numpy as jnp
from jax.experimental import pallas as pl
from jax.experimental.pallas import tpu as pltpu

BN = 2048


def _body(x_ref, e_ref, dist_ref, ind_ref, q_ref):
    xb = x_ref[...]
    e = e_ref[...]
    d = jax.lax.dot_general(xb, e, (((1,), (1,)), ((), ())),
                            preferred_element_type=jnp.float32)
    dist_ref[...] = d
    idx = jnp.argmax(d, axis=-1).astype(jnp.int32)
    ind_ref[0, 0, :] = idx
    oh = (jax.lax.broadcasted_iota(jnp.int32, d.shape, 1) == idx[:, None]
          ).astype(jnp.float32)
    q_ref[...] = jax.lax.dot_general(oh, e, (((1,), (0,)), ((), ())),
                                     preferred_element_type=jnp.float32)


def kernel(x, embed):
    x = x.astype(jnp.float32)
    b, n, d = x.shape
    h, c, _ = embed.shape
    N = b * n
    xf = x.reshape(N, d)
    ef = embed.reshape(c, d)
    grid = (N // BN,)
    dist, ind3, quant = pl.pallas_call(
        _body,
        grid=grid,
        in_specs=[
            pl.BlockSpec((BN, d), lambda i: (i, 0)),
            pl.BlockSpec((c, d), lambda i: (0, 0)),
        ],
        out_specs=[
            pl.BlockSpec((BN, c), lambda i: (i, 0)),
            pl.BlockSpec((1, 1, BN), lambda i: (i, 0, 0)),
            pl.BlockSpec((BN, d), lambda i: (i, 0)),
        ],
        out_shape=[
            jax.ShapeDtypeStruct((N, c), jnp.float32),
            jax.ShapeDtypeStruct((N // BN, 1, BN), jnp.int32),
            jax.ShapeDtypeStruct((N, d), jnp.float32),
        ],
    )(xf, ef)
    quantize = quant.reshape(b, n, d)
    embed_ind = ind3.reshape(b, n)
    dist_out = dist.reshape(h, b, n, c)
    return quantize, embed_ind, dist_out
